# baseline (speedup 1.0000x reference)
import functools

import jax
import jax.numpy as jnp
from jax import lax
from jax.experimental import pallas as pl
from jax.experimental.pallas import tpu as pltpu
from jax.experimental.pallas import tpu_sc as plsc

BATCH = 16384
HIDDEN = 64
CHUNK = 128
NUM_WORKERS = 32
CPW = BATCH // (NUM_WORKERS * CHUNK)
NUM_CHUNKS = BATCH // CHUNK


def _build():
    mesh = plsc.VectorSubcoreMesh(core_axis_name="c", subcore_axis_name="s")

    @functools.partial(
        pl.kernel,
        mesh=mesh,
        out_type=jax.ShapeDtypeStruct((NUM_CHUNKS, CHUNK, HIDDEN), jnp.float32),
        scratch_types=[
            pltpu.VMEM((CPW, CHUNK), jnp.int32),
            pltpu.VMEM((CPW, CHUNK, HIDDEN), jnp.float32),
            pltpu.SemaphoreType.DMA,
        ],
        compiler_params=pltpu.CompilerParams(use_tc_tiling_on_sc=False),
    )
    def emb(idx_hbm, table_hbm, out_hbm, idx_v, rows_v, sem):
        wid = lax.axis_index("s") * 2 + lax.axis_index("c")
        base = wid * CPW
        pltpu.sync_copy(idx_hbm.at[pl.ds(base, CPW)], idx_v)
        handles = [
            pltpu.async_copy(table_hbm.at[idx_v.at[j]], rows_v.at[j], sem)
            for j in range(CPW)
        ]
        for h in handles:
            h.wait()
        pltpu.sync_copy(rows_v, out_hbm.at[pl.ds(base, CPW)])

    return emb


_EMB = _build()


def kernel(labels, embedding_table):
    idx = labels.astype(jnp.int32).reshape(NUM_CHUNKS, CHUNK)
    out = _EMB(idx, embedding_table)
    return out.reshape(BATCH, HIDDEN)


# speedup vs baseline: 1.0010x; 1.0010x over previous
import functools

import jax
import jax.numpy as jnp
from jax import lax
from jax.experimental import pallas as pl
from jax.experimental.pallas import tpu as pltpu
from jax.experimental.pallas import tpu_sc as plsc

BATCH = 16384
HIDDEN = 64
CHUNK = 128
NUM_WORKERS = 32
ROWS_PER_WORKER = BATCH // NUM_WORKERS
CPW = ROWS_PER_WORKER // CHUNK


def _build():
    mesh = plsc.VectorSubcoreMesh(core_axis_name="c", subcore_axis_name="s")

    @functools.partial(
        pl.kernel,
        mesh=mesh,
        out_type=jax.ShapeDtypeStruct((BATCH, HIDDEN), jnp.float32),
        scratch_types=[
            pltpu.VMEM((ROWS_PER_WORKER,), jnp.int32),
            pltpu.VMEM((ROWS_PER_WORKER, HIDDEN), jnp.float32),
            pltpu.SemaphoreType.DMA,
        ],
        compiler_params=pltpu.CompilerParams(use_tc_tiling_on_sc=False),
    )
    def emb(idx_hbm, table_hbm, out_hbm, idx_v, rows_v, sem):
        wid = lax.axis_index("s") * 2 + lax.axis_index("c")
        base = wid * ROWS_PER_WORKER
        pltpu.sync_copy(idx_hbm.at[pl.ds(base, ROWS_PER_WORKER)], idx_v)
        handles = [
            pltpu.async_copy(
                table_hbm.at[idx_v.at[pl.ds(j * CHUNK, CHUNK)]],
                rows_v.at[pl.ds(j * CHUNK, CHUNK)],
                sem,
            )
            for j in range(CPW)
        ]
        for h in handles:
            h.wait()
        pltpu.sync_copy(rows_v, out_hbm.at[pl.ds(base, ROWS_PER_WORKER)])

    return emb


_EMB = _build()


def kernel(labels, embedding_table):
    return _EMB(labels.astype(jnp.int32), embedding_table)


# speedup vs baseline: 1.7202x; 1.7185x over previous
import functools

import jax
import jax.numpy as jnp
from jax import lax
from jax.experimental import pallas as pl
from jax.experimental.pallas import tpu as pltpu
from jax.experimental.pallas import tpu_sc as plsc

BATCH = 16384
HIDDEN = 64
NUM_WORKERS = 32
ROWS_PER_WORKER = BATCH // NUM_WORKERS


def _build():
    mesh = plsc.VectorSubcoreMesh(core_axis_name="c", subcore_axis_name="s")

    @functools.partial(
        pl.kernel,
        mesh=mesh,
        out_type=jax.ShapeDtypeStruct((BATCH, HIDDEN), jnp.float32),
        scratch_types=[
            pltpu.VMEM((ROWS_PER_WORKER,), jnp.int32),
            pltpu.SMEM((ROWS_PER_WORKER,), jnp.int32),
            pltpu.VMEM((ROWS_PER_WORKER, HIDDEN), jnp.float32),
            pltpu.SemaphoreType.DMA,
        ],
    )
    def emb(idx_hbm, table_hbm, out_hbm, idx_v, idx_s, rows_v, sem):
        wid = lax.axis_index("s") * 2 + lax.axis_index("c")
        base = wid * ROWS_PER_WORKER
        pltpu.sync_copy(idx_hbm.at[pl.ds(base, ROWS_PER_WORKER)], idx_v)

        def body(g, carry):
            off = pl.multiple_of(g * 16, 16)
            vec = idx_v[pl.ds(off, 16)]
            for k in range(16):
                i = vec[k]
                pltpu.async_copy(
                    table_hbm.at[pl.ds(i, 1)],
                    rows_v.at[pl.ds(off + k, 1)],
                    sem,
                )
            return carry

        lax.fori_loop(0, ROWS_PER_WORKER // 16, body, 0)
        pltpu.make_async_copy(
            table_hbm.at[pl.ds(0, ROWS_PER_WORKER)], rows_v, sem
        ).wait()
        pltpu.sync_copy(rows_v, out_hbm.at[pl.ds(base, ROWS_PER_WORKER)])

    return emb


_EMB = _build()


def kernel(labels, embedding_table):
    return _EMB(labels.astype(jnp.int32), embedding_table)


# speedup vs baseline: 2.3515x; 1.3670x over previous
import functools

import jax
import jax.numpy as jnp
from jax import lax
from jax.experimental import pallas as pl
from jax.experimental.pallas import tpu as pltpu
from jax.experimental.pallas import tpu_sc as plsc

BATCH = 16384
HIDDEN = 64
BLK = 128
NUM_WORKERS = 32
RPW = BATCH // NUM_WORKERS
SLABS = RPW // BLK
GPS = BLK // 16
WAVE = 4


def _build():
    mesh = plsc.VectorSubcoreMesh(core_axis_name="c", subcore_axis_name="s")

    @functools.partial(
        pl.kernel,
        mesh=mesh,
        out_type=jax.ShapeDtypeStruct((HIDDEN, BATCH), jnp.float32),
        scratch_types=[
            pltpu.VMEM((RPW,), jnp.int32),
            pltpu.VMEM((2, WAVE, HIDDEN, BLK), jnp.float32),
            pltpu.VMEM((HIDDEN, BLK), jnp.float32),
            pltpu.SemaphoreType.DMA,
            pltpu.SemaphoreType.DMA,
        ],
        compiler_params=pltpu.CompilerParams(needs_layout_passes=False),
    )
    def emb(idx_hbm, tabt_hbm, out_hbm, idx_v, blocks_v, slab_v, sem0, sem1):
        wid = lax.axis_index("s") * 2 + lax.axis_index("c")
        base = wid * RPW
        pltpu.sync_copy(idx_hbm.at[pl.ds(base, RPW)], idx_v)
        sems = (sem0, sem1)
        rows4 = [
            jax.lax.iota(jnp.int32, 16) + (16 * q) for q in range(HIDDEN // 16)
        ]

        def fire(vec, wv):
            slot = wv % 2
            for t in range(WAVE):
                i = vec[WAVE * wv + t]
                lane = lax.rem(i, BLK)
                blk = pl.multiple_of(i - lane, BLK)
                pltpu.async_copy(
                    tabt_hbm.at[:, pl.ds(blk, BLK)],
                    blocks_v.at[slot, t],
                    sems[slot],
                )

        def drain_extract(vec, wv, col0):
            slot = wv % 2
            for t in range(WAVE):
                pltpu.make_async_copy(
                    tabt_hbm.at[:, pl.ds(0, BLK)],
                    blocks_v.at[slot, t],
                    sems[slot],
                ).wait()
            for t in range(WAVE):
                i = vec[WAVE * wv + t]
                lane_v = jnp.full((16,), lax.rem(i, BLK), jnp.int32)
                col_v = jnp.full((16,), col0 + WAVE * wv + t, jnp.int32)
                block = blocks_v.at[slot, t]
                for q in range(HIDDEN // 16):
                    vals = plsc.load_gather(block, [rows4[q], lane_v])
                    plsc.store_scatter(slab_v, [rows4[q], col_v], vals)

        def body(g, carry):
            col0 = g * 16
            off = pl.multiple_of(carry + col0, 16)
            vec = idx_v[pl.ds(off, 16)]
            fire(vec, 0)
            fire(vec, 1)
            drain_extract(vec, 0, col0)
            fire(vec, 2)
            drain_extract(vec, 1, col0)
            fire(vec, 3)
            drain_extract(vec, 2, col0)
            drain_extract(vec, 3, col0)
            return carry

        for m in range(SLABS):
            lax.fori_loop(0, GPS, body, m * BLK)
            pltpu.sync_copy(
                slab_v, out_hbm.at[:, pl.ds(base + m * BLK, BLK)]
            )

    return emb


_EMB = _build()


def kernel(labels, embedding_table):
    table_t = jnp.swapaxes(embedding_table, 0, 1)
    out_t = _EMB(labels.astype(jnp.int32), table_t)
    return jnp.swapaxes(out_t, 0, 1)


# speedup vs baseline: 3.4639x; 1.4730x over previous
import functools

import jax
import jax.numpy as jnp
from jax import lax
from jax.experimental import pallas as pl
from jax.experimental.pallas import tpu as pltpu
from jax.experimental.pallas import tpu_sc as plsc

BATCH = 16384
HIDDEN = 64
BLK = 128
NUM_WORKERS = 32
RPW = BATCH // NUM_WORKERS
GROUPS = RPW // 16
NBUF = 8
CSTG = 16 * HIDDEN


def _build():
    mesh = plsc.VectorSubcoreMesh(core_axis_name="c", subcore_axis_name="s")

    @functools.partial(
        pl.kernel,
        mesh=mesh,
        out_type=jax.ShapeDtypeStruct((BATCH * HIDDEN,), jnp.float32),
        scratch_types=[
            pltpu.VMEM((RPW,), jnp.int32),
            pltpu.VMEM((RPW,), jnp.int32),
            pltpu.VMEM((RPW,), jnp.int32),
            pltpu.VMEM((RPW,), jnp.int32),
            pltpu.VMEM((NBUF * HIDDEN, BLK), jnp.float32),
            pltpu.VMEM((2 * CSTG,), jnp.float32),
            pltpu.SemaphoreType.DMA,
            pltpu.SemaphoreType.DMA,
            pltpu.SemaphoreType.DMA,
            pltpu.SemaphoreType.DMA,
        ],
        compiler_params=pltpu.CompilerParams(needs_layout_passes=False),
    )
    def emb(sidx_hbm, pos_hbm, newf_hbm, did_hbm, tabt_hbm, out_hbm,
            sidx_v, pos_v, newf_v, did_v, ring_v, cstg_v,
            sem0, sem1, semout0, semout1):
        wid = lax.axis_index("s") * 2 + lax.axis_index("c")
        base = wid * RPW
        pltpu.sync_copy(sidx_hbm.at[pl.ds(base, RPW)], sidx_v)
        pltpu.sync_copy(pos_hbm.at[pl.ds(base, RPW)], pos_v)
        pltpu.sync_copy(newf_hbm.at[pl.ds(base, RPW)], newf_v)
        pltpu.sync_copy(did_hbm.at[pl.ds(base, RPW)], did_v)
        sems = (sem0, sem1)
        did0 = did_v[pl.ds(0, 16)][0]
        rows4 = [
            jax.lax.iota(jnp.int32, 16) + (16 * q) for q in range(HIDDEN // 16)
        ]

        def fire(sv, nv, dv, u):
            for k in range(4 * u, 4 * u + 4):
                sk = sv[k]
                lk = sk & 127
                ck = pl.multiple_of(sk - lk, BLK)
                s64 = pl.multiple_of(((dv[k] - did0) & (NBUF - 1)) * HIDDEN,
                                     HIDDEN)

                @pl.when(nv[k] != 0)
                def _():
                    pltpu.async_copy(
                        tabt_hbm.at[:, pl.ds(ck, BLK)],
                        ring_v.at[pl.ds(s64, HIDDEN), :],
                        sems[u % 2],
                    )

        semouts = (semout0, semout1)

        def drain_extract(sv, pv, nv, dv, u, cpar, par):
            for k in range(4 * u, 4 * u + 4):
                s64 = pl.multiple_of(((dv[k] - did0) & (NBUF - 1)) * HIDDEN,
                                     HIDDEN)

                @pl.when(nv[k] != 0)
                def _():
                    pltpu.make_async_copy(
                        tabt_hbm.at[:, pl.ds(0, BLK)],
                        ring_v.at[pl.ds(s64, HIDDEN), :],
                        sems[u % 2],
                    ).wait()
            for k in range(4 * u, 4 * u + 4):
                sk = sv[k]
                lk = sk & 127
                s64 = ((dv[k] - did0) & (NBUF - 1)) * HIDDEN
                lane_v = jnp.full((16,), lk, jnp.int32)
                kk = k & 15
                for q in range(HIDDEN // 16):
                    vals = plsc.load_gather(ring_v, [rows4[q] + s64, lane_v])
                    cstg_v[pl.ds(pl.multiple_of(cpar + kk * HIDDEN + 16 * q,
                                                16), 16)] = vals
                po = pl.multiple_of(pv[k] * HIDDEN, HIDDEN)
                pltpu.async_copy(
                    cstg_v.at[pl.ds(pl.multiple_of(cpar + kk * HIDDEN, HIDDEN),
                                    HIDDEN)],
                    out_hbm.at[pl.ds(po, HIDDEN)],
                    semouts[par],
                )

        def process_group(g, par):
            g16 = pl.multiple_of(g * 16, 16)
            sv = sidx_v[pl.ds(g16, 16)]
            pv = pos_v[pl.ds(g16, 16)]
            nv = newf_v[pl.ds(g16, 16)]
            dv = did_v[pl.ds(g16, 16)]
            cpar = par * CSTG
            fire(sv, nv, dv, 0)
            fire(sv, nv, dv, 1)
            drain_extract(sv, pv, nv, dv, 0, cpar, par)
            fire(sv, nv, dv, 2)
            drain_extract(sv, pv, nv, dv, 1, cpar, par)
            fire(sv, nv, dv, 3)
            drain_extract(sv, pv, nv, dv, 2, cpar, par)
            drain_extract(sv, pv, nv, dv, 3, cpar, par)

        def drain_out(par):
            pltpu.make_async_copy(
                out_hbm.at[pl.ds(0, CSTG)],
                cstg_v.at[pl.ds(par * CSTG, CSTG)],
                semouts[par],
            ).wait()

        def body(t, carry):
            @pl.when(t >= 1)
            def _():
                drain_out(0)

            process_group(2 * t, 0)

            @pl.when(t >= 1)
            def _():
                drain_out(1)

            process_group(2 * t + 1, 1)
            return carry

        lax.fori_loop(0, GROUPS // 2, body, 0)
        drain_out(0)
        drain_out(1)

    return emb


_EMB = _build()


def kernel(labels, embedding_table):
    idx32 = labels.astype(jnp.int32)
    order = jnp.argsort(idx32).astype(jnp.int32)
    sidx = jnp.take(idx32, order)
    blk = sidx >> 7
    first = (jnp.arange(BATCH, dtype=jnp.int32) % RPW) == 0
    shifted = jnp.concatenate(
        [jnp.ones((1,), jnp.bool_), blk[1:] != blk[:-1]]
    )
    newf = (first | shifted).astype(jnp.int32)
    did = jnp.cumsum(newf).astype(jnp.int32) - 1
    table_t = jnp.swapaxes(embedding_table, 0, 1)
    out1d = _EMB(sidx, order, newf, did, table_t)
    return out1d.reshape(BATCH, HIDDEN)
